# baseline (speedup 1.0000x reference)
import functools

import jax
import jax.numpy as jnp
from jax import lax
from jax.experimental import pallas as pl
from jax.experimental.pallas import tpu as pltpu
from jax.experimental.pallas import tpu_sc as plsc

_NC = 2
_NS = 16
_NW = _NC * _NS

_HOP = 256
_CHUNK = 16384
_FPC = _CHUNK // _HOP
_RPC = _FPC + 1


def _make_sc_call(batch, seq_len):
    n_chunks = seq_len // _CHUNK

    @functools.partial(
        pl.kernel,
        out_type=jax.ShapeDtypeStruct((batch, seq_len), jnp.float32),
        mesh=plsc.VectorSubcoreMesh(
            core_axis_name="c", subcore_axis_name="s",
            num_cores=_NC, num_subcores=_NS),
        scratch_types=[
            pltpu.VMEM((_HOP,), jnp.float32),
            pltpu.VMEM((_RPC * _HOP,), jnp.float32),
            pltpu.VMEM((_RPC * _HOP,), jnp.float32),
            pltpu.VMEM((_CHUNK,), jnp.float32),
            pltpu.VMEM((_CHUNK,), jnp.float32),
            pltpu.VMEM((_CHUNK,), jnp.float32),
            pltpu.VMEM((_CHUNK,), jnp.float32),
            pltpu.SemaphoreType.DMA,
            pltpu.SemaphoreType.DMA,
            pltpu.SemaphoreType.DMA,
            pltpu.SemaphoreType.DMA,
            pltpu.SemaphoreType.DMA,
            pltpu.SemaphoreType.DMA,
        ],
        compiler_params=pltpu.CompilerParams(needs_layout_passes=False),
    )
    def sc_call(wp_hbm, tab_hbm, p2_hbm, out_hbm,
                p2_v, rows_a, rows_b, wp_a, wp_b, out_a, out_b,
                sem_rows_a, sem_rows_b, sem_in_a, sem_in_b,
                sem_out_a, sem_out_b):
        wid = lax.axis_index("s") * _NC + lax.axis_index("c")
        rows_bufs = (rows_a, rows_b)
        wp_bufs = (wp_a, wp_b)
        out_bufs = (out_a, out_b)
        sem_rows = (sem_rows_a, sem_rows_b)
        sem_in = (sem_in_a, sem_in_b)
        sem_out = (sem_out_a, sem_out_b)

        def issue_chunk(c, buf):
            cps = [pltpu.async_copy(
                wp_hbm.at[wid, pl.ds(c * _CHUNK, _CHUNK)],
                wp_bufs[buf], sem_in[buf])]
            for r in range(_RPC):
                cps.append(pltpu.async_copy(
                    tab_hbm.at[c * _FPC + r, wid, :],
                    rows_bufs[buf].at[pl.ds(r * _HOP, _HOP)],
                    sem_rows[buf]))
            return cps

        pltpu.sync_copy(p2_hbm, p2_v)
        pend = [None, None]
        pend[0] = issue_chunk(0, 0)
        out_cp = [None, None]

        for c in range(n_chunks):
            buf = c & 1
            if c + 1 < n_chunks:
                pend[1 - buf] = issue_chunk(c + 1, 1 - buf)
            for cp in pend[buf]:
                cp.wait()
            if c >= 2:
                out_cp[buf].wait()
            wp_v = wp_bufs[buf]
            out_v = out_bufs[buf]
            rows_v = rows_bufs[buf]

            @plsc.parallel_loop(0, _CHUNK // 16, unroll=2)
            def _grp(k, wp_v=wp_v, out_v=out_v, rows_v=rows_v):
                off = k * 16
                base = lax.shift_right_logical(k, 4) * _HOP
                tab_f = rows_v.at[pl.ds(base, 2 * _HOP)]
                wpv = wp_v[pl.ds(off, 16)]
                p2 = p2_v[pl.ds(jnp.bitwise_and(k, 15) * 16, 16)]
                raw = wpv * jnp.float32(_HOP)
                fi = raw.astype(jnp.int32)
                p = raw - fi.astype(jnp.float32)
                i01 = jnp.bitwise_and(fi + 1, _HOP - 1)
                a = plsc.load_gather(tab_f, [fi])
                b = plsc.load_gather(tab_f, [i01])
                cc = plsc.load_gather(tab_f, [fi + _HOP])
                dd = plsc.load_gather(tab_f, [i01 + _HOP])
                low = a + p * (b - a)
                high = cc + p * (dd - cc)
                out_v[pl.ds(off, 16)] = low + p2 * (high - low)

            out_cp[buf] = pltpu.async_copy(
                out_v, out_hbm.at[wid, pl.ds(c * _CHUNK, _CHUNK)],
                sem_out[buf])
        out_cp[0].wait()
        out_cp[1].wait()

    return sc_call


def kernel(wrapped_phase, tables, hop_length):
    batch, seq_len = wrapped_phase.shape
    frames = seq_len // _HOP
    assert seq_len % _CHUNK == 0 and batch == _NW
    assert tables.shape == (batch, frames + 1, _HOP)

    tab_t = jnp.transpose(tables, (1, 0, 2))
    p2row = jnp.arange(_HOP, dtype=jnp.float32) / jnp.asarray(
        hop_length, jnp.float32)

    sc_call = _make_sc_call(batch, seq_len)
    return sc_call(wrapped_phase, tab_t, p2row)


# speedup vs baseline: 1.0220x; 1.0220x over previous
import functools

import jax
import jax.numpy as jnp
from jax import lax
from jax.experimental import pallas as pl
from jax.experimental.pallas import tpu as pltpu
from jax.experimental.pallas import tpu_sc as plsc

_NC = 2
_NS = 16
_NW = _NC * _NS

_HOP = 256
_CHUNK = 16384
_FPC = _CHUNK // _HOP
_RPC = _FPC + 1


def _make_sc_call(batch, seq_len):
    n_chunks = seq_len // _CHUNK

    @functools.partial(
        pl.kernel,
        out_type=jax.ShapeDtypeStruct((batch, seq_len), jnp.float32),
        mesh=plsc.VectorSubcoreMesh(
            core_axis_name="c", subcore_axis_name="s",
            num_cores=_NC, num_subcores=_NS),
        scratch_types=[
            pltpu.VMEM((_HOP,), jnp.float32),
            pltpu.VMEM((_RPC * _HOP,), jnp.float32),
            pltpu.VMEM((_RPC * _HOP,), jnp.float32),
            pltpu.VMEM((_CHUNK,), jnp.float32),
            pltpu.VMEM((_CHUNK,), jnp.float32),
            pltpu.VMEM((_CHUNK,), jnp.float32),
            pltpu.VMEM((_CHUNK,), jnp.float32),
            pltpu.SemaphoreType.DMA,
            pltpu.SemaphoreType.DMA,
            pltpu.SemaphoreType.DMA,
            pltpu.SemaphoreType.DMA,
            pltpu.SemaphoreType.DMA,
            pltpu.SemaphoreType.DMA,
        ],
        compiler_params=pltpu.CompilerParams(needs_layout_passes=False),
    )
    def sc_call(wp_hbm, tab_hbm, p2_hbm, out_hbm,
                p2_v, rows_a, rows_b, wp_a, wp_b, out_a, out_b,
                sem_rows_a, sem_rows_b, sem_in_a, sem_in_b,
                sem_out_a, sem_out_b):
        wid = lax.axis_index("s") * _NC + lax.axis_index("c")
        rows_bufs = (rows_a, rows_b)
        wp_bufs = (wp_a, wp_b)
        out_bufs = (out_a, out_b)
        sem_rows = (sem_rows_a, sem_rows_b)
        sem_in = (sem_in_a, sem_in_b)
        sem_out = (sem_out_a, sem_out_b)

        def issue_chunk(c, buf):
            cps = [pltpu.async_copy(
                wp_hbm.at[wid, pl.ds(c * _CHUNK, _CHUNK)],
                wp_bufs[buf], sem_in[buf])]
            for r in range(_RPC):
                cps.append(pltpu.async_copy(
                    tab_hbm.at[c * _FPC + r, wid, :],
                    rows_bufs[buf].at[pl.ds(r * _HOP, _HOP)],
                    sem_rows[buf]))
            return cps

        pltpu.sync_copy(p2_hbm, p2_v)
        pend = [None, None]
        pend[0] = issue_chunk(0, 0)
        out_cp = [None, None]

        for c in range(n_chunks):
            buf = c & 1
            if c + 1 < n_chunks:
                pend[1 - buf] = issue_chunk(c + 1, 1 - buf)
            for cp in pend[buf]:
                cp.wait()
            if c >= 2:
                out_cp[buf].wait()
            wp_v = wp_bufs[buf]
            out_v = out_bufs[buf]
            rows_v = rows_bufs[buf]

            @plsc.parallel_loop(0, _CHUNK // 16, unroll=4)
            def _grp(k, wp_v=wp_v, out_v=out_v, rows_v=rows_v):
                off = k * 16
                base = lax.shift_right_logical(k, 4) * _HOP
                tab_f = rows_v.at[pl.ds(base, 2 * _HOP)]
                wpv = wp_v[pl.ds(off, 16)]
                p2 = p2_v[pl.ds(jnp.bitwise_and(k, 15) * 16, 16)]
                raw = wpv * jnp.float32(_HOP)
                fi = raw.astype(jnp.int32)
                p = raw - fi.astype(jnp.float32)
                i01 = jnp.bitwise_and(fi + 1, _HOP - 1)
                a = plsc.load_gather(tab_f, [fi])
                b = plsc.load_gather(tab_f, [i01])
                cc = plsc.load_gather(tab_f, [fi + _HOP])
                dd = plsc.load_gather(tab_f, [i01 + _HOP])
                low = a + p * (b - a)
                high = cc + p * (dd - cc)
                out_v[pl.ds(off, 16)] = low + p2 * (high - low)

            out_cp[buf] = pltpu.async_copy(
                out_v, out_hbm.at[wid, pl.ds(c * _CHUNK, _CHUNK)],
                sem_out[buf])
        out_cp[0].wait()
        out_cp[1].wait()

    return sc_call


def kernel(wrapped_phase, tables, hop_length):
    batch, seq_len = wrapped_phase.shape
    frames = seq_len // _HOP
    assert seq_len % _CHUNK == 0 and batch == _NW
    assert tables.shape == (batch, frames + 1, _HOP)

    tab_t = jnp.transpose(tables, (1, 0, 2))
    p2row = jnp.arange(_HOP, dtype=jnp.float32) / jnp.asarray(
        hop_length, jnp.float32)

    sc_call = _make_sc_call(batch, seq_len)
    return sc_call(wrapped_phase, tab_t, p2row)
